# baseline (speedup 1.0000x reference)
import functools

import jax
import jax.numpy as jnp
from jax import lax
from jax.experimental import pallas as pl
from jax.experimental.pallas import tpu as pltpu
from jax.experimental.pallas import tpu_sc as plsc
from jax._src.pallas import mpmd

_NC, _NS = 2, 16
_NW = _NC * _NS


def _make_sc_copy(max_len, d_model, tec_rows, tec_chunk, scs_chunk):
    scs_rows = max_len - tec_rows
    rows_per_w = tec_rows // _NW
    nchunk = rows_per_w // tec_chunk
    rows_per_scs = scs_rows // _NC
    n_scs_chunk = rows_per_scs // scs_chunk

    scalar_mesh = plsc.ScalarSubcoreMesh(axis_name="c")
    vector_mesh = plsc.VectorSubcoreMesh(core_axis_name="c", subcore_axis_name="s")

    def scs_fn(emb_hbm, out_hbm, tile_buf, spmem_buf):
        del tile_buf
        base = tec_rows + lax.axis_index("c") * rows_per_scs
        for k in range(n_scs_chunk):
            src = pl.ds(base + k * scs_chunk, scs_chunk)
            pltpu.sync_copy(emb_hbm.at[src], spmem_buf)
            pltpu.sync_copy(spmem_buf, out_hbm.at[src])

    def tec_fn(emb_hbm, out_hbm, tile_buf, spmem_buf):
        del spmem_buf
        wid = lax.axis_index("s") * _NC + lax.axis_index("c")
        base = wid * rows_per_w
        for k in range(nchunk):
            src = pl.ds(base + k * tec_chunk, tec_chunk)
            pltpu.sync_copy(emb_hbm.at[src], tile_buf)
            pltpu.sync_copy(tile_buf, out_hbm.at[src])

    return mpmd.mpmd_map(
        [(scalar_mesh, scs_fn), (vector_mesh, tec_fn)],
        out_types=jax.ShapeDtypeStruct((max_len, d_model), jnp.float32),
        scratch_types=[
            (pltpu.MemorySpace.VMEM @ vector_mesh)(
                (tec_chunk, d_model), jnp.float32),
            pltpu.VMEM_SHARED((scs_chunk, d_model), jnp.float32),
        ],
    )


def kernel(x, embedding):
    seq_len = x.shape[1]
    max_len, d_model = embedding.shape
    sc_copy = _make_sc_copy(max_len, d_model,
                            tec_rows=5120, tec_chunk=80, scs_chunk=768)
    out = sc_copy(embedding)
    return out[None, :seq_len, :]
